# baseline (speedup 1.0000x reference)
import functools

import jax
import jax.numpy as jnp
from jax import lax
from jax.experimental import pallas as pl
from jax.experimental.pallas import tpu as pltpu
from jax.experimental.pallas import tpu_sc as plsc

L = 16
NC = 2
NS = 16
NW = NC * NS
CHUNK = 80


def _dist_mult_body(D, NCHUNK, h_hbm, src_hbm, dst_hbm, rel_hbm, w_hbm,
                    out_hbm, sidx, didx, relv, ubuf, vbuf, rbuf, sbuf, obuf,
                    sem0, sem1):
    wid = lax.axis_index("s") * NC + lax.axis_index("c")
    pltpu.sync_copy(src_hbm.at[wid], sidx)
    pltpu.sync_copy(dst_hbm.at[wid], didx)
    pltpu.sync_copy(rel_hbm.at[wid], relv)

    sems = (sem0, sem1)
    iota16 = lax.iota(jnp.int32, L)
    lastmask = iota16 == (L - 1)

    def fire(c, b):
        pltpu.make_async_copy(
            h_hbm.at[sidx.at[c]], ubuf.at[b], sems[b]).start()
        pltpu.make_async_copy(
            h_hbm.at[didx.at[c]], vbuf.at[b], sems[b]).start()
        pltpu.make_async_copy(
            w_hbm.at[relv.at[c]], rbuf.at[b], sems[b]).start()

    def wait(c, b):
        pltpu.make_async_copy(
            h_hbm.at[sidx.at[c]], ubuf.at[b], sems[b]).wait()
        pltpu.make_async_copy(
            h_hbm.at[didx.at[c]], vbuf.at[b], sems[b]).wait()
        pltpu.make_async_copy(
            w_hbm.at[relv.at[c]], rbuf.at[b], sems[b]).wait()

    def compute(c, b):
        wait(c, b)
        ub = ubuf.at[b]
        vb = vbuf.at[b]
        rb = rbuf.at[b]

        def group_body(g, gcarry):
            for l in range(L):
                e = g * L + l
                acc0 = jnp.zeros((L,), jnp.float32)
                acc1 = jnp.zeros((L,), jnp.float32)
                for j in range(D // L):
                    u = ub[e, pl.ds(j * L, L)]
                    v = vb[e, pl.ds(j * L, L)]
                    r = rb[e, pl.ds(j * L, L)]
                    if j % 2 == 0:
                        acc0 = acc0 + u * v * r
                    else:
                        acc1 = acc1 + u * v * r
                csum = plsc.cumsum(acc0 + acc1)
                plsc.store_compressed(
                    sbuf.at[pl.ds(e, L)], csum, mask=lastmask)
            return gcarry

        lax.fori_loop(0, CHUNK // L, group_body, 0)

        def sig_body(g, gcarry):
            s = sbuf[pl.ds(g * L, L)]
            obuf[c, pl.ds(g * L, L)] = 1.0 / (1.0 + jnp.exp(-s))
            return gcarry

        lax.fori_loop(0, CHUNK // L, sig_body, 0)

    fire(0, 0)

    def superstep(s, carry):
        c0 = 2 * s
        fire(c0 + 1, 1)
        compute(c0, 0)
        fire(c0 + 2, 0)
        compute(c0 + 1, 1)
        return carry

    lax.fori_loop(0, (NCHUNK - 1) // 2, superstep, 0)
    compute(NCHUNK - 1, 0)

    pltpu.sync_copy(obuf, out_hbm.at[wid])


def kernel(h, edge_index, rel_ids, W):
    E = edge_index.shape[1]
    D = h.shape[1]
    EPW = E // NW
    NCHUNK = EPW // CHUNK
    assert EPW * NW == E and NCHUNK * CHUNK == EPW and NCHUNK % 2 == 1

    src = edge_index[0].reshape(NW, NCHUNK, CHUNK)
    dst = edge_index[1].reshape(NW, NCHUNK, CHUNK)
    rel = rel_ids.reshape(NW, NCHUNK, CHUNK)

    mesh = plsc.VectorSubcoreMesh(core_axis_name="c", subcore_axis_name="s")
    sc_kernel = functools.partial(
        pl.kernel,
        mesh=mesh,
        compiler_params=pltpu.CompilerParams(needs_layout_passes=False),
        out_type=jax.ShapeDtypeStruct((NW, NCHUNK, CHUNK), jnp.float32),
        scratch_types=[
            pltpu.VMEM((NCHUNK, CHUNK), jnp.int32),
            pltpu.VMEM((NCHUNK, CHUNK), jnp.int32),
            pltpu.VMEM((NCHUNK, CHUNK), jnp.int32),
            pltpu.VMEM((2, CHUNK, D), jnp.float32),
            pltpu.VMEM((2, CHUNK, D), jnp.float32),
            pltpu.VMEM((2, CHUNK, D), jnp.float32),
            pltpu.VMEM((CHUNK + L,), jnp.float32),
            pltpu.VMEM((NCHUNK, CHUNK), jnp.float32),
            pltpu.SemaphoreType.DMA,
            pltpu.SemaphoreType.DMA,
        ],
    )(functools.partial(_dist_mult_body, D, NCHUNK))
    out = sc_kernel(h, src, dst, rel, W)
    return out.reshape(E)


# speedup vs baseline: 6.8878x; 6.8878x over previous
import functools

import jax
import jax.numpy as jnp
from jax import lax
from jax.experimental import pallas as pl
from jax.experimental.pallas import tpu as pltpu
from jax.experimental.pallas import tpu_sc as plsc

L = 16
NC = 2
NS = 16
NW = NC * NS
CHUNK = 80


def _dist_mult_body(D, NCHUNK, h_hbm, src_hbm, dst_hbm, rel_hbm, w_hbm,
                    out_hbm, sidx, didx, relv, ubuf, vbuf, wloc, sbuf, obuf,
                    sem0, sem1):
    wid = lax.axis_index("s") * NC + lax.axis_index("c")
    pltpu.sync_copy(w_hbm, wloc)
    pltpu.sync_copy(src_hbm.at[wid], sidx)
    pltpu.sync_copy(dst_hbm.at[wid], didx)
    pltpu.sync_copy(rel_hbm.at[wid], relv)

    sems = (sem0, sem1)
    iota16 = lax.iota(jnp.int32, L)
    lastmask = iota16 == (L - 1)
    wflat = wloc.reshape(1, wloc.shape[0] * D)
    zero16 = jnp.zeros((L,), jnp.int32)
    wcols = [j * L + iota16 for j in range(D // L)]

    def fire(c, b):
        pltpu.make_async_copy(
            h_hbm.at[sidx.at[c]], ubuf.at[b], sems[b]).start()
        pltpu.make_async_copy(
            h_hbm.at[didx.at[c]], vbuf.at[b], sems[b]).start()

    def wait(c, b):
        pltpu.make_async_copy(
            h_hbm.at[sidx.at[c]], ubuf.at[b], sems[b]).wait()
        pltpu.make_async_copy(
            h_hbm.at[didx.at[c]], vbuf.at[b], sems[b]).wait()

    def compute(c, b):
        wait(c, b)
        ub = ubuf.at[b]
        vb = vbuf.at[b]

        def group_body(g, gcarry):
            relD16 = relv[c, pl.ds(g * L, L)] * D
            for l in range(L):
                e = g * L + l
                m = jnp.where(iota16 == l, relD16, 0)
                bcast = (plsc.cumsum(m)
                         + lax.rev(plsc.cumsum(lax.rev(m, (0,))), (0,)) - m)
                acc0 = jnp.zeros((L,), jnp.float32)
                acc1 = jnp.zeros((L,), jnp.float32)
                for j in range(D // L):
                    u = ub[e, pl.ds(j * L, L)]
                    v = vb[e, pl.ds(j * L, L)]
                    r = plsc.load_gather(wflat, [zero16, bcast + wcols[j]])
                    if j % 2 == 0:
                        acc0 = acc0 + u * v * r
                    else:
                        acc1 = acc1 + u * v * r
                csum = plsc.cumsum(acc0 + acc1)
                plsc.store_compressed(
                    sbuf.at[pl.ds(e, L)], csum, mask=lastmask)
            return gcarry

        lax.fori_loop(0, CHUNK // L, group_body, 0)

        def sig_body(g, gcarry):
            s = sbuf[pl.ds(g * L, L)]
            obuf[c, pl.ds(g * L, L)] = 1.0 / (1.0 + jnp.exp(-s))
            return gcarry

        lax.fori_loop(0, CHUNK // L, sig_body, 0)

    fire(0, 0)

    def superstep(s, carry):
        c0 = 2 * s
        fire(c0 + 1, 1)
        compute(c0, 0)
        fire(c0 + 2, 0)
        compute(c0 + 1, 1)
        return carry

    lax.fori_loop(0, (NCHUNK - 1) // 2, superstep, 0)
    compute(NCHUNK - 1, 0)

    pltpu.sync_copy(obuf, out_hbm.at[wid])


def kernel(h, edge_index, rel_ids, W):
    E = edge_index.shape[1]
    D = h.shape[1]
    EPW = E // NW
    NCHUNK = EPW // CHUNK
    assert EPW * NW == E and NCHUNK * CHUNK == EPW and NCHUNK % 2 == 1

    src = edge_index[0].reshape(NW, NCHUNK, CHUNK)
    dst = edge_index[1].reshape(NW, NCHUNK, CHUNK)
    rel = rel_ids.reshape(NW, NCHUNK, CHUNK)

    mesh = plsc.VectorSubcoreMesh(core_axis_name="c", subcore_axis_name="s")
    sc_kernel = functools.partial(
        pl.kernel,
        mesh=mesh,
        compiler_params=pltpu.CompilerParams(needs_layout_passes=False),
        out_type=jax.ShapeDtypeStruct((NW, NCHUNK, CHUNK), jnp.float32),
        scratch_types=[
            pltpu.VMEM((NCHUNK, CHUNK), jnp.int32),
            pltpu.VMEM((NCHUNK, CHUNK), jnp.int32),
            pltpu.VMEM((NCHUNK, CHUNK), jnp.int32),
            pltpu.VMEM((2, CHUNK, D), jnp.float32),
            pltpu.VMEM((2, CHUNK, D), jnp.float32),
            pltpu.VMEM((W.shape[0], D), jnp.float32),
            pltpu.VMEM((CHUNK + L,), jnp.float32),
            pltpu.VMEM((NCHUNK, CHUNK), jnp.float32),
            pltpu.SemaphoreType.DMA,
            pltpu.SemaphoreType.DMA,
        ],
    )(functools.partial(_dist_mult_body, D, NCHUNK))
    out = sc_kernel(h, src, dst, rel, W)
    return out.reshape(E)


# speedup vs baseline: 7.2515x; 1.0528x over previous
import functools

import jax
import jax.numpy as jnp
from jax import lax
from jax.experimental import pallas as pl
from jax.experimental.pallas import tpu as pltpu
from jax.experimental.pallas import tpu_sc as plsc

L = 16
NC = 2
NS = 16
NW = NC * NS
CHUNK = 80


def _dist_mult_body(D, NCHUNK, h_hbm, src_hbm, dst_hbm, rel_hbm, w_hbm,
                    out_hbm, sidx, didx, relv, ubuf, vbuf, wloc, sbuf, obuf,
                    sem0, sem1):
    wid = lax.axis_index("s") * NC + lax.axis_index("c")
    pltpu.sync_copy(w_hbm, wloc)
    pltpu.sync_copy(src_hbm.at[wid], sidx)
    pltpu.sync_copy(dst_hbm.at[wid], didx)
    pltpu.sync_copy(rel_hbm.at[wid], relv)

    sems = (sem0, sem1)
    iota16 = lax.iota(jnp.int32, L)
    lastmask = iota16 == (L - 1)
    wflat = wloc
    zero16 = jnp.zeros((L,), jnp.int32)
    wcols = [j * L + iota16 for j in range(D // L)]

    def fire(c, b):
        pltpu.make_async_copy(
            h_hbm.at[sidx.at[c]], ubuf.at[b], sems[b]).start()
        pltpu.make_async_copy(
            h_hbm.at[didx.at[c]], vbuf.at[b], sems[b]).start()

    def wait(c, b):
        pltpu.make_async_copy(
            h_hbm.at[sidx.at[c]], ubuf.at[b], sems[b]).wait()
        pltpu.make_async_copy(
            h_hbm.at[didx.at[c]], vbuf.at[b], sems[b]).wait()

    def compute(c, b):
        wait(c, b)
        ub = ubuf.at[b]
        vb = vbuf.at[b]

        def group_body(g, gcarry):
            relD16 = relv[c, pl.ds(g * L, L)] * D
            for l in range(L):
                e = g * L + l
                m = jnp.where(iota16 == l, relD16, 0)
                bcast = (plsc.cumsum(m)
                         + lax.rev(plsc.cumsum(lax.rev(m, (0,))), (0,)) - m)
                acc0 = jnp.zeros((L,), jnp.float32)
                acc1 = jnp.zeros((L,), jnp.float32)
                for j in range(D // (2 * L)):
                    u2 = plsc.bitcast(ub[e, pl.ds(j * L, L)], jnp.bfloat16)
                    v2 = plsc.bitcast(vb[e, pl.ds(j * L, L)], jnp.bfloat16)
                    ua, ubb = plsc.unpack(u2, format=plsc.PackFormat.INTERLEAVED)
                    va, vbb = plsc.unpack(v2, format=plsc.PackFormat.INTERLEAVED)
                    ra = plsc.load_gather(
                        wflat, [zero16, bcast + wcols[2 * j]])
                    rb = plsc.load_gather(
                        wflat, [zero16, bcast + wcols[2 * j + 1]])
                    acc0 = acc0 + ua * va * ra
                    acc1 = acc1 + ubb * vbb * rb
                csum = plsc.cumsum(acc0 + acc1)
                plsc.store_compressed(
                    sbuf.at[pl.ds(e, L)], csum, mask=lastmask)
            return gcarry

        lax.fori_loop(0, CHUNK // L, group_body, 0)

        def sig_body(g, gcarry):
            s = sbuf[pl.ds(g * L, L)]
            obuf[c, pl.ds(g * L, L)] = 1.0 / (1.0 + jnp.exp(-s))
            return gcarry

        lax.fori_loop(0, CHUNK // L, sig_body, 0)

    fire(0, 0)

    def superstep(s, carry):
        c0 = 2 * s
        fire(c0 + 1, 1)
        compute(c0, 0)
        fire(c0 + 2, 0)
        compute(c0 + 1, 1)
        return carry

    lax.fori_loop(0, (NCHUNK - 1) // 2, superstep, 0)
    compute(NCHUNK - 1, 0)

    pltpu.sync_copy(obuf, out_hbm.at[wid])


def kernel(h, edge_index, rel_ids, W):
    E = edge_index.shape[1]
    D = h.shape[1]
    EPW = E // NW
    NCHUNK = EPW // CHUNK
    assert EPW * NW == E and NCHUNK * CHUNK == EPW and NCHUNK % 2 == 1

    src = edge_index[0].reshape(NW, NCHUNK, CHUNK)
    dst = edge_index[1].reshape(NW, NCHUNK, CHUNK)
    rel = rel_ids.reshape(NW, NCHUNK, CHUNK)

    h16 = lax.bitcast_convert_type(
        h.astype(jnp.bfloat16).reshape(h.shape[0], D // 2, 2), jnp.int32)
    cols = jnp.arange(D).reshape(D // 32, 16, 2).transpose(0, 2, 1).reshape(D)
    Wp = W[:, cols].reshape(1, W.shape[0] * D)

    mesh = plsc.VectorSubcoreMesh(core_axis_name="c", subcore_axis_name="s")
    sc_kernel = functools.partial(
        pl.kernel,
        mesh=mesh,
        compiler_params=pltpu.CompilerParams(
            needs_layout_passes=False, use_tc_tiling_on_sc=False),
        out_type=jax.ShapeDtypeStruct((NW, NCHUNK, CHUNK), jnp.float32),
        scratch_types=[
            pltpu.VMEM((NCHUNK, CHUNK), jnp.int32),
            pltpu.VMEM((NCHUNK, CHUNK), jnp.int32),
            pltpu.VMEM((NCHUNK, CHUNK), jnp.int32),
            pltpu.VMEM((2, CHUNK, D // 2), jnp.int32),
            pltpu.VMEM((2, CHUNK, D // 2), jnp.int32),
            pltpu.VMEM((1, W.shape[0] * D), jnp.float32),
            pltpu.VMEM((CHUNK + L,), jnp.float32),
            pltpu.VMEM((NCHUNK, CHUNK), jnp.float32),
            pltpu.SemaphoreType.DMA,
            pltpu.SemaphoreType.DMA,
        ],
    )(functools.partial(_dist_mult_body, D, NCHUNK))
    out = sc_kernel(h16, src, dst, rel, Wp)
    return out.reshape(E)


# speedup vs baseline: 7.3897x; 1.0191x over previous
import functools

import jax
import jax.numpy as jnp
from jax import lax
from jax.experimental import pallas as pl
from jax.experimental.pallas import tpu as pltpu
from jax.experimental.pallas import tpu_sc as plsc

L = 16
NC = 2
NS = 16
NW = NC * NS
CHUNK = 80


def _dist_mult_body(D, NCHUNK, h_hbm, src_hbm, dst_hbm, rel_hbm, w_hbm,
                    out_hbm, sidx, didx, relv, ubuf, vbuf, wloc, sbuf,
                    obuf, sem0, sem1):
    wid = lax.axis_index("s") * NC + lax.axis_index("c")
    pltpu.sync_copy(w_hbm, wloc)
    pltpu.sync_copy(src_hbm.at[wid], sidx)
    pltpu.sync_copy(dst_hbm.at[wid], didx)
    pltpu.sync_copy(rel_hbm.at[wid], relv)

    sems = (sem0, sem1)
    iota16 = lax.iota(jnp.int32, L)
    lastmask = iota16 == (L - 1)
    wflat = wloc
    zero16 = jnp.zeros((L,), jnp.int32)
    wcols = [j * L + iota16 for j in range(D // L)]

    def fire(c, b):
        pltpu.make_async_copy(
            h_hbm.at[sidx.at[c]], ubuf.at[b], sems[b]).start()
        pltpu.make_async_copy(
            h_hbm.at[didx.at[c]], vbuf.at[b], sems[b]).start()

    def wait(c, b):
        pltpu.make_async_copy(
            h_hbm.at[sidx.at[c]], ubuf.at[b], sems[b]).wait()
        pltpu.make_async_copy(
            h_hbm.at[didx.at[c]], vbuf.at[b], sems[b]).wait()

    def compute(c, b):
        wait(c, b)
        ub = ubuf.at[b]
        vb = vbuf.at[b]

        def group_body(g, gcarry):
            relD16 = relv[c, pl.ds(g * L, L)] * D
            for l in range(L):
                e = g * L + l
                m = jnp.where(iota16 == l, relD16, 0)
                bcast = (plsc.cumsum(m)
                         + lax.rev(plsc.cumsum(lax.rev(m, (0,))), (0,)) - m)
                acc0 = jnp.zeros((L,), jnp.float32)
                acc1 = jnp.zeros((L,), jnp.float32)
                for j in range(D // (2 * L)):
                    u2 = plsc.bitcast(ub[e, pl.ds(j * L, L)], jnp.bfloat16)
                    v2 = plsc.bitcast(vb[e, pl.ds(j * L, L)], jnp.bfloat16)
                    uv = u2 * v2
                    pa, pb = plsc.unpack(uv, format=plsc.PackFormat.INTERLEAVED)
                    ra = plsc.load_gather(
                        wflat, [zero16, bcast + wcols[2 * j]])
                    rc = plsc.load_gather(
                        wflat, [zero16, bcast + wcols[2 * j + 1]])
                    acc0 = acc0 + pa * ra
                    acc1 = acc1 + pb * rc
                csum = plsc.cumsum(acc0 + acc1)
                plsc.store_compressed(
                    sbuf.at[pl.ds(e, L)], csum, mask=lastmask)
            return gcarry

        lax.fori_loop(0, CHUNK // L, group_body, 0)

        def sig_body(g, gcarry):
            s = sbuf[pl.ds(g * L, L)]
            obuf[c, pl.ds(g * L, L)] = 1.0 / (1.0 + jnp.exp(-s))
            return gcarry

        lax.fori_loop(0, CHUNK // L, sig_body, 0)

    fire(0, 0)

    def superstep(s, carry):
        c0 = 2 * s
        fire(c0 + 1, 1)
        compute(c0, 0)
        fire(c0 + 2, 0)
        compute(c0 + 1, 1)
        return carry

    lax.fori_loop(0, (NCHUNK - 1) // 2, superstep, 0)
    compute(NCHUNK - 1, 0)

    pltpu.sync_copy(obuf, out_hbm.at[wid])


def kernel(h, edge_index, rel_ids, W):
    E = edge_index.shape[1]
    D = h.shape[1]
    EPW = E // NW
    NCHUNK = EPW // CHUNK
    assert EPW * NW == E and NCHUNK * CHUNK == EPW and NCHUNK % 2 == 1

    src = edge_index[0].reshape(NW, NCHUNK, CHUNK)
    dst = edge_index[1].reshape(NW, NCHUNK, CHUNK)
    rel = rel_ids.reshape(NW, NCHUNK, CHUNK)

    h16 = lax.bitcast_convert_type(
        h.astype(jnp.bfloat16).reshape(h.shape[0], D // 2, 2), jnp.int32)
    cols = jnp.arange(D).reshape(D // 32, 16, 2).transpose(0, 2, 1).reshape(D)
    Wp = W[:, cols].reshape(1, W.shape[0] * D)

    mesh = plsc.VectorSubcoreMesh(core_axis_name="c", subcore_axis_name="s")
    sc_kernel = functools.partial(
        pl.kernel,
        mesh=mesh,
        compiler_params=pltpu.CompilerParams(
            needs_layout_passes=False, use_tc_tiling_on_sc=False),
        out_type=jax.ShapeDtypeStruct((NW, NCHUNK, CHUNK), jnp.float32),
        scratch_types=[
            pltpu.VMEM((NCHUNK, CHUNK), jnp.int32),
            pltpu.VMEM((NCHUNK, CHUNK), jnp.int32),
            pltpu.VMEM((NCHUNK, CHUNK), jnp.int32),
            pltpu.VMEM((2, CHUNK, D // 2), jnp.int32),
            pltpu.VMEM((2, CHUNK, D // 2), jnp.int32),
            pltpu.VMEM((1, W.shape[0] * D), jnp.float32),
            pltpu.VMEM((CHUNK + L,), jnp.float32),
            pltpu.VMEM((NCHUNK, CHUNK), jnp.float32),
            pltpu.SemaphoreType.DMA,
            pltpu.SemaphoreType.DMA,
        ],
    )(functools.partial(_dist_mult_body, D, NCHUNK))
    out = sc_kernel(h16, src, dst, rel, Wp)
    return out.reshape(E)
